# baseline (speedup 1.0000x reference)
import functools

import jax
import jax.numpy as jnp
from jax import lax
from jax.experimental import pallas as pl
from jax.experimental.pallas import tpu as pltpu
from jax.experimental.pallas import tpu_sc as plsc

N = 10000
E = 320000
B = 128
EMB = 128
NUM_LAYER = 5

NC = 2
NS = 16
NTILES = NC * NS
OWN = 320
ROWS = NTILES * OWN

TRASH = NS * OWN
E_TOT = E + N
SLABS = 162
EPAD = SLABS * 16 * 128
CAPCH = EPAD // 128

_f32 = jnp.float32
_i32 = jnp.int32


def _part_body(src_hbm, dst_hbm, cid_hbm, bsrc_hbm, bdst_hbm, bcid_hbm,
               cnt_hbm, ssrc, sdst, scid, bufs, bufd, bufc, cvec):
    c = lax.axis_index("c")
    s = lax.axis_index("s")
    tid = c * NS + s
    lo = tid * OWN
    hi = lo + OWN
    base = c * (NS * OWN)

    iota16 = lax.iota(_i32, 16)

    def _init(i, _):
        bufs[pl.ds(i * 16, 16)] = jnp.zeros((16,), _i32)
        bufd[pl.ds(i * 16, 16)] = jnp.full((16,), TRASH, _i32)
        bufc[pl.ds(i * 16, 16)] = jnp.full((16,), 18, _i32)
        return 0

    lax.fori_loop(0, 10, _init, 0)

    def slab(bi, carry):
        pltpu.sync_copy(src_hbm.at[pl.ds(bi * 16, 16)], ssrc)
        pltpu.sync_copy(dst_hbm.at[pl.ds(bi * 16, 16)], sdst)
        pltpu.sync_copy(cid_hbm.at[pl.ds(bi * 16, 16)], scid)

        def step(k, carry2):
            off, gctr = carry2
            r = k // 8
            j = k % 8
            sl = pl.ds(j * 16, 16)
            dv = sdst[r, sl]
            m = (dv >= lo) & (dv < hi)
            keys = jnp.where(m, iota16, iota16 + 16)
            ks, sc_ = plsc.sort_key_val(keys, ssrc[r, sl])
            _, dc_ = plsc.sort_key_val(keys, dv)
            _, cc_ = plsc.sort_key_val(keys, scid[r, sl])
            good = ks < 16
            bufs[pl.ds(off, 16)] = jnp.where(good, sc_, 0)
            bufd[pl.ds(off, 16)] = jnp.where(good, dc_ - base, TRASH)
            bufc[pl.ds(off, 16)] = jnp.where(good, cc_, 18)
            off = off + plsc.all_reduce_population_count(m)[0]
            do = off >= 128

            @pl.when(do)
            def _flush():
                row = tid * CAPCH + gctr
                pltpu.sync_copy(bufs.at[pl.ds(0, 128)], bsrc_hbm.at[row])
                pltpu.sync_copy(bufd.at[pl.ds(0, 128)], bdst_hbm.at[row])
                pltpu.sync_copy(bufc.at[pl.ds(0, 128)], bcid_hbm.at[row])
                for bb in (bufs, bufd, bufc):
                    tail = bb[pl.ds(128, 16)]
                    bb[pl.ds(0, 16)] = tail

            off = jnp.where(do, off - 128, off)
            gctr = jnp.where(do, gctr + 1, gctr)
            return off, gctr

        return lax.fori_loop(0, 128, step, carry)

    off, gctr = lax.fori_loop(0, SLABS, slab, (jnp.int32(0), jnp.int32(0)))

    @pl.when(off > 0)
    def _final_flush():
        for kk in range(8):
            bufs[pl.ds(off + kk * 16, 16)] = jnp.zeros((16,), _i32)
            bufd[pl.ds(off + kk * 16, 16)] = jnp.full((16,), TRASH, _i32)
            bufc[pl.ds(off + kk * 16, 16)] = jnp.full((16,), 18, _i32)
        row = tid * CAPCH + gctr
        pltpu.sync_copy(bufs.at[pl.ds(0, 128)], bsrc_hbm.at[row])
        pltpu.sync_copy(bufd.at[pl.ds(0, 128)], bdst_hbm.at[row])
        pltpu.sync_copy(bufc.at[pl.ds(0, 128)], bcid_hbm.at[row])

    cvec[...] = jnp.broadcast_to(gctr * 128 + off, (16,))
    pltpu.sync_copy(cvec, cnt_hbm.at[tid])


@functools.cache
def _part_kernel():
    return functools.partial(
        pl.kernel,
        out_type=[
            jax.ShapeDtypeStruct((NTILES * CAPCH, 128), _i32),
            jax.ShapeDtypeStruct((NTILES * CAPCH, 128), _i32),
            jax.ShapeDtypeStruct((NTILES * CAPCH, 128), _i32),
            jax.ShapeDtypeStruct((NTILES, 16), _i32),
        ],
        mesh=plsc.VectorSubcoreMesh(core_axis_name="c", subcore_axis_name="s",
                                    num_cores=NC, num_subcores=NS),
        compiler_params=pltpu.CompilerParams(needs_layout_passes=False),
        scratch_types=[
            pltpu.VMEM((16, 128), _i32),
            pltpu.VMEM((16, 128), _i32),
            pltpu.VMEM((16, 128), _i32),
            pltpu.VMEM((160,), _i32),
            pltpu.VMEM((160,), _i32),
            pltpu.VMEM((160,), _i32),
            pltpu.VMEM((16,), _i32),
        ],
    )(_part_body)


def _acc_body(h_hbm, etab_hbm, bsrc_hbm, bdst_hbm, bcid_hbm, cnt_hbm, out_hbm,
              srcv, dstv, cidv, rows_v, erows_v, acc_sh, cvec, sem):
    c = lax.axis_index("c")
    s = lax.axis_index("s")
    tid = c * NS + s

    def _zero(i, _):
        rows_v[i // 8, pl.ds((i % 8) * 16, 16)] = jnp.zeros((16,), _f32)
        return 0

    lax.fori_loop(0, 128 * 8, _zero, 0)
    pltpu.sync_copy(rows_v, acc_sh.at[pl.ds(s * OWN, 128)])
    pltpu.sync_copy(rows_v, acc_sh.at[pl.ds(s * OWN + 128, 128)])
    pltpu.sync_copy(rows_v.at[pl.ds(0, 64)], acc_sh.at[pl.ds(s * OWN + 256, 64)])

    @pl.when(s == 0)
    def _zero_trash():
        pltpu.sync_copy(rows_v, acc_sh.at[pl.ds(NS * OWN, 128)])

    plsc.subcore_barrier()

    pltpu.sync_copy(cnt_hbm.at[tid], cvec)
    cnt = cvec[...][0]
    nch = (cnt + 127) // 128

    def chunk(gi, _):
        row = tid * CAPCH + gi
        pltpu.sync_copy(bsrc_hbm.at[row], srcv)
        pltpu.sync_copy(bdst_hbm.at[row], dstv)
        pltpu.sync_copy(bcid_hbm.at[row], cidv)
        pltpu.async_copy(h_hbm.at[srcv], rows_v, sem).wait()
        pltpu.async_copy(etab_hbm.at[cidv], erows_v, sem).wait()

        def addm(k, _2):
            r = k // 8
            sl = pl.ds((k % 8) * 16, 16)
            rows_v[r, sl] = rows_v[r, sl] + erows_v[r, sl]
            return 0

        lax.fori_loop(0, 128 * 8, addm, 0)
        pltpu.sync_copy(rows_v, acc_sh.at[dstv], add=True)
        return 0

    lax.fori_loop(0, nch, chunk, 0)
    plsc.subcore_barrier()
    pltpu.sync_copy(acc_sh.at[pl.ds(s * OWN, 128)],
                    out_hbm.at[pl.ds(tid * OWN, 128)])
    pltpu.sync_copy(acc_sh.at[pl.ds(s * OWN + 128, 128)],
                    out_hbm.at[pl.ds(tid * OWN + 128, 128)])
    pltpu.sync_copy(acc_sh.at[pl.ds(s * OWN + 256, 64)],
                    out_hbm.at[pl.ds(tid * OWN + 256, 64)])


@functools.cache
def _acc_kernel():
    return functools.partial(
        pl.kernel,
        out_type=jax.ShapeDtypeStruct((ROWS, EMB), _f32),
        mesh=plsc.VectorSubcoreMesh(core_axis_name="c", subcore_axis_name="s",
                                    num_cores=NC, num_subcores=NS),
        scratch_types=[
            pltpu.VMEM((128,), _i32),
            pltpu.VMEM((128,), _i32),
            pltpu.VMEM((128,), _i32),
            pltpu.VMEM((128, EMB), _f32),
            pltpu.VMEM((128, EMB), _f32),
            pltpu.VMEM_SHARED((NS * OWN + 128, EMB), _f32),
            pltpu.VMEM((16,), _i32),
            pltpu.SemaphoreType.DMA,
        ],
    )(_acc_body)


def _h0_body(x0, x1, a1, a2, o_ref):
    acc = jnp.zeros((N, EMB), _f32)
    for k in range(3):
        m0 = (x0[...] == k).astype(_f32)
        m1 = (x1[...] == k).astype(_f32)
        acc = acc + m0 * a1[k:k + 1, :] + m1 * a2[k:k + 1, :]
    o_ref[...] = acc


def _redwin(scr, ncols, square):
    def win(lo):
        def body(i, a):
            sl = scr[pl.ds(lo + i * 8, 8), :]
            if square:
                sl = sl * sl
            return a + sl
        a = lax.fori_loop(0, 625, body, jnp.zeros((8, ncols), _f32))
        a4 = a[0:4] + a[4:8]
        a2 = a4[0:2] + a4[2:4]
        return a2[0:1] + a2[1:2]
    return (win(0) + win(5000)) * jnp.float32(1e-4)


def _mlp_body(aggin, w1, b1, w2, b2, gam, bet, o_ref, scr, *, relu):
    agg = aggin[0:N, :]
    bf16 = jnp.bfloat16
    t = jnp.maximum(jnp.dot(agg.astype(bf16), w1[...].astype(bf16),
                            preferred_element_type=_f32) + b1[...], 0.0)
    h2 = jnp.dot(t.astype(bf16), w2[...].astype(bf16),
                 preferred_element_type=_f32) + b2[...]
    scr[...] = h2
    mu = _redwin(scr, EMB, False)
    d = h2 - mu
    scr[...] = d
    var = _redwin(scr, EMB, True)
    hn = d / jnp.sqrt(var + 1e-5) * gam[...] + bet[...]
    if relu:
        hn = jnp.maximum(hn, 0.0)
    o_ref[...] = hn


def _final_body(n1, n2, bt1, bt2, ci1, ci2, o_ref):
    hi = lax.Precision.HIGHEST
    rows = lax.broadcasted_iota(_i32, (B, 1), 0)
    cols = lax.broadcasted_iota(_i32, (1, N), 1)
    oh1 = (bt1[...] == rows).astype(_f32)
    oh2 = (bt2[...] == rows).astype(_f32)
    s1 = jnp.dot(oh1, n1[...], preferred_element_type=_f32, precision=hi)
    s2 = jnp.dot(oh2, n2[...], preferred_element_type=_f32, precision=hi)
    c1 = jnp.maximum(jnp.sum(oh1, axis=1, keepdims=True), 1.0)
    c2 = jnp.maximum(jnp.sum(oh2, axis=1, keepdims=True), 1.0)
    p1 = s1 / c1
    p2 = s2 / c2
    ohc1 = (ci1[...] == cols).astype(_f32)
    ohc2 = (ci2[...] == cols).astype(_f32)
    ce1 = jnp.dot(ohc1, n1[...], preferred_element_type=_f32, precision=hi)
    ce2 = jnp.dot(ohc2, n2[...], preferred_element_type=_f32, precision=hi)
    o_ref[...] = jnp.sum(p1 * p2 + ce1 * ce2, axis=1)[None, :]


_h0_call = pl.pallas_call(_h0_body, out_shape=jax.ShapeDtypeStruct((N, EMB), _f32))
_mlp_scratch = [pltpu.VMEM((N, EMB), _f32)]
_mlp_call_relu = pl.pallas_call(functools.partial(_mlp_body, relu=True),
                                out_shape=jax.ShapeDtypeStruct((N, EMB), _f32),
                                scratch_shapes=_mlp_scratch)
_mlp_call_norelu = pl.pallas_call(functools.partial(_mlp_body, relu=False),
                                  out_shape=jax.ShapeDtypeStruct((N, EMB), _f32),
                                  scratch_shapes=_mlp_scratch)
_final_call = pl.pallas_call(_final_body,
                             out_shape=jax.ShapeDtypeStruct((1, B), _f32))


def _prep_edges(edge_index, edge_attr):
    loop = jnp.arange(N, dtype=_i32)
    pad = EPAD - E_TOT
    src = jnp.concatenate([edge_index[0], loop, jnp.zeros((pad,), _i32)])
    dst = jnp.concatenate([edge_index[1], loop, jnp.full((pad,), N, _i32)])
    cid = jnp.concatenate([edge_attr[:, 0] * 3 + edge_attr[:, 1],
                           jnp.full((N,), 12, _i32),
                           jnp.full((pad,), 18, _i32)])
    shape = (EPAD // 128, 128)
    return src.reshape(shape), dst.reshape(shape), cid.reshape(shape)


def kernel(x1, edge_index1, edge_attr1, batch1, center_node_idx1,
           x2, edge_index2, edge_attr2, batch2, center_node_idx2,
           atom_emb1, atom_emb2, edge_emb1, edge_emb2,
           W1, b1, W2, b2, bn_g, bn_b):
    a1 = atom_emb1[:8]
    a2 = jnp.concatenate([atom_emb2, jnp.zeros((5, EMB), _f32)], axis=0)
    i0 = jnp.arange(18, dtype=_i32) // 3
    i1 = jnp.arange(18, dtype=_i32) % 3
    etab = edge_emb1[:, i0, :] + edge_emb2[:, i1, :]
    etab = jnp.concatenate([etab, jnp.zeros((NUM_LAYER, 6, EMB), _f32)], 1)

    def graph_rep_nodes(x, edge_index, edge_attr):
        src2d, dst2d, cid2d = _prep_edges(edge_index, edge_attr)
        bsrc, bdst, bcid, cnt = _part_kernel()(src2d, dst2d, cid2d)
        h = _h0_call(x[:, 0:1], x[:, 1:2], a1, a2)
        for l in range(NUM_LAYER):
            agg = _acc_kernel()(h, etab[l], bsrc, bdst, bcid, cnt)
            call = _mlp_call_norelu if l == NUM_LAYER - 1 else _mlp_call_relu
            h = call(agg, W1[l], b1[l][None, :], W2[l], b2[l][None, :],
                     bn_g[l][None, :], bn_b[l][None, :])
        return h

    n1 = graph_rep_nodes(x1, edge_index1, edge_attr1)
    n2 = graph_rep_nodes(x2, edge_index2, edge_attr2)
    sim = _final_call(n1, n2, batch1[None, :], batch2[None, :],
                      center_node_idx1[:, None], center_node_idx2[:, None])
    return sim[0]
